# Initial kernel scaffold; baseline (speedup 1.0000x reference)
import jax
import jax.numpy as jnp
from jax.experimental import pallas as pl


def kernel(heatmap):
    raise NotImplementedError("write your pallas kernel here")


# speedup vs baseline: 27.1281x; 27.1281x over previous
import jax
import jax.numpy as jnp
from jax import lax
from jax.experimental import pallas as pl
from jax.experimental.pallas import tpu as pltpu

_TEMP = 0.5
_K = 256
_R = 6912
_L = 128


def _row_kernel(x_ref, o_ref, keys_ref):
    x = x_ref[0]
    u = lax.bitcast_convert_type(x, jnp.uint32)
    sign = u >> jnp.uint32(31)
    keys = u ^ (sign * jnp.uint32(0x7FFFFFFF) + jnp.uint32(0x80000000))
    keys_ref[...] = keys

    n_total = jnp.int32(_R * _L)

    def cond(c):
        i, p, cnt = c
        return jnp.logical_and(i < 32, cnt != _K)

    def body(c):
        i, p, cnt = c
        bit = jnp.uint32(0x80000000) >> i.astype(jnp.uint32)
        cand = p | bit
        cn = jnp.sum((keys_ref[...] >= cand).astype(jnp.int32))
        take = cn >= _K
        return (i + 1,
                jnp.where(take, cand, p),
                jnp.where(take, cn, cnt))

    _, p, cnt = lax.while_loop(
        cond, body, (jnp.int32(0), jnp.uint32(0), n_total))

    keys = keys_ref[...]
    mask_ge = keys >= p
    mask_eq = keys == p
    n_eq = jnp.sum(mask_eq.astype(jnp.int32))
    n_gt = cnt - n_eq
    frac = (jnp.int32(_K) - n_gt).astype(jnp.float32) / jnp.maximum(
        n_eq, 1).astype(jnp.float32)
    sel = jnp.where(mask_eq, frac, jnp.where(mask_ge, 1.0, 0.0))

    m = jnp.max(x)
    w = jnp.exp((x - m) * (1.0 / _TEMP)) * sel

    r = lax.broadcasted_iota(jnp.int32, (_R, _L), 0)
    cidx = lax.broadcasted_iota(jnp.int32, (_R, _L), 1)
    d = r // 72
    rem = (r - d * 72) * _L + cidx
    h = rem // 96
    wc = rem - h * 96

    den = jnp.sum(w) + 1e-20
    od = jnp.sum(w * d.astype(jnp.float32)) / den
    oh = jnp.sum(w * h.astype(jnp.float32)) / den
    ow = jnp.sum(w * wc.astype(jnp.float32)) / den

    lane = lax.broadcasted_iota(jnp.int32, (1, _L), 1)
    out = jnp.where(lane == 0, od,
                    jnp.where(lane == 1, oh,
                              jnp.where(lane == 2, ow, 0.0)))
    o_ref[0] = out


def kernel(heatmap):
    B, C, D, H, W = heatmap.shape
    x = heatmap.reshape(B * C, _R, _L)
    out = pl.pallas_call(
        _row_kernel,
        grid=(B * C,),
        in_specs=[pl.BlockSpec((1, _R, _L), lambda i: (i, 0, 0))],
        out_specs=pl.BlockSpec((1, 1, _L), lambda i: (i, 0, 0)),
        out_shape=jax.ShapeDtypeStruct((B * C, 1, _L), jnp.float32),
        scratch_shapes=[pltpu.VMEM((_R, _L), jnp.uint32)],
    )(x)
    return out[:, 0, :3].reshape(B, C, 3)


# speedup vs baseline: 39.2856x; 1.4482x over previous
import functools

import jax
import jax.numpy as jnp
from jax import lax
from jax.experimental import pallas as pl
from jax.experimental.pallas import tpu as pltpu
from jax.experimental.pallas import tpu_sc as plsc

_TEMP = 0.5
_K = 256
_N = 96 * 96 * 96
_ROWS = 64
_W = 32768
_NWIN = _N // _W
_CHUNK = 512
_NCHUNK = _W // _CHUNK
_CAP = 2048
_NCV = _CAP // 16
_IMIN = -2147483648


def _key_of(v):
    u = lax.bitcast_convert_type(v, jnp.int32)
    return jnp.where(u >= 0, u, u ^ jnp.int32(0x7FFFFFFF))


def _val_of_key(k):
    u = jnp.where(k >= 0, k, k ^ jnp.int32(0x7FFFFFFF))
    return lax.bitcast_convert_type(u, jnp.float32)


def _sc_kernel(x_hbm, o_hbm, win, ck, ci, outv, sptr, skey, sthr, smax):
    lanes = lax.iota(jnp.int32, 16)

    def count_ge(cand):
        def cbody(i, acc):
            kv = ck[pl.ds(i * 16, 16)]
            return acc + jnp.where(kv >= cand, 1, 0).astype(jnp.int32)
        acc = lax.fori_loop(0, _NCV, cbody, jnp.zeros((16,), jnp.int32))
        return jnp.sum(acc)

    def kth_key():
        def body(i, c):
            p, cnt = c
            cand = p + (jnp.int32(1) << (31 - i))
            cn = count_ge(cand)
            take = jnp.logical_and(cnt != _K, cn >= _K)
            return (jnp.where(take, cand, p), jnp.where(take, cn, cnt))

        p, _ = lax.fori_loop(0, 32, body, (jnp.int32(_IMIN), jnp.int32(-1)))
        return p

    def reselect():
        tnew = kth_key()
        skey[0] = tnew
        tf = jnp.max(_val_of_key(jnp.full((16,), tnew, jnp.int32)))
        sthr[0] = tf
        def comp(i, wp):
            kv = ck[pl.ds(i * 16, 16)]
            iv = ci[pl.ds(i * 16, 16)]
            msk = kv >= tnew
            plsc.store_compressed(ck.at[pl.ds(wp, 16)], kv, mask=msk)
            plsc.store_compressed(ci.at[pl.ds(wp, 16)], iv, mask=msk)
            return wp + jnp.sum(jnp.where(msk, 1, 0).astype(jnp.int32))
        wp = lax.fori_loop(0, _NCV, comp, jnp.int32(0))
        sptr[0] = wp
        def clr(i, _):
            pos = i * 16 + lanes
            kv = ck[pl.ds(i * 16, 16)]
            ck[pl.ds(i * 16, 16)] = jnp.where(pos >= wp, _IMIN, kv)
            return 0
        lax.fori_loop(0, _NCV, clr, jnp.int32(0))

    def do_row(row):
        def init(i, _):
            ck[pl.ds(i * 16, 16)] = jnp.full((16,), _IMIN, jnp.int32)
            return 0
        lax.fori_loop(0, _NCV, init, jnp.int32(0))
        sptr[0] = jnp.int32(0)
        skey[0] = jnp.int32(_IMIN)
        sthr[0] = jnp.float32(-jnp.inf)
        smax[0] = jnp.float32(-jnp.inf)

        def window(w, _):
            pltpu.sync_copy(x_hbm.at[row, pl.ds(w * _W, _W)], win)

            def chunk(c, _):
                @pl.when(sptr[0] > _CAP - (_CHUNK + 8))
                def _():
                    reselect()

                base = c * _CHUNK
                mv = win[pl.ds(base, 16)]
                for j in range(1, 32):
                    mv = jnp.maximum(mv, win[pl.ds(base + j * 16, 16)])
                cmax = jnp.max(mv)
                smax[0] = jnp.maximum(smax[0], cmax)

                @pl.when(cmax >= sthr[0])
                def _():
                    tkey = skey[0]
                    ptr = sptr[0]
                    gbase = w * _W + base
                    for j in range(32):
                        v = win[pl.ds(base + j * 16, 16)]
                        kv = _key_of(v)
                        msk = kv >= tkey
                        plsc.store_compressed(ck.at[pl.ds(ptr, 16)], kv, mask=msk)
                        plsc.store_compressed(
                            ci.at[pl.ds(ptr, 16)], gbase + j * 16 + lanes,
                            mask=msk)
                        ptr = ptr + jnp.sum(
                            jnp.where(msk, 1, 0).astype(jnp.int32))
                    sptr[0] = ptr
                return 0

            lax.fori_loop(0, _NCHUNK, chunk, jnp.int32(0))
            return 0

        lax.fori_loop(0, _NWIN, window, jnp.int32(0))

        tstar = kth_key()
        m = smax[0]

        def wbody(i, accs):
            dgt, ddt, dht, dwt, deq, det, het, wet, ngt, neq = accs
            kv = ck[pl.ds(i * 16, 16)]
            iv = ci[pl.ds(i * 16, 16)]
            gt = kv > tstar
            eq = kv == tstar
            ge = kv >= tstar
            vv = _val_of_key(kv)
            e = jnp.where(ge, jnp.exp((vv - m) * (1.0 / _TEMP)),
                          jnp.float32(0.0))
            d = (iv // 9216).astype(jnp.float32)
            rem = iv - (iv // 9216) * 9216
            h = (rem // 96).astype(jnp.float32)
            wc = (rem - (rem // 96) * 96).astype(jnp.float32)
            egt = jnp.where(gt, e, 0.0)
            eeq = jnp.where(eq, e, 0.0)
            return (dgt + egt, ddt + egt * d, dht + egt * h, dwt + egt * wc,
                    deq + eeq, det + eeq * d, het + eeq * h, wet + eeq * wc,
                    ngt + jnp.where(gt, 1, 0).astype(jnp.int32),
                    neq + jnp.where(eq, 1, 0).astype(jnp.int32))

        z = jnp.zeros((16,), jnp.float32)
        zi = jnp.zeros((16,), jnp.int32)
        accs = lax.fori_loop(0, _NCV, wbody,
                             (z, z, z, z, z, z, z, z, zi, zi))
        dgt, ddt, dht, dwt, deq, det, het, wet, ngt, neq = accs
        n_gt = jnp.sum(ngt)
        n_eq = jnp.sum(neq)
        fv = (jnp.full((16,), jnp.int32(_K) - n_gt, jnp.int32)
              .astype(jnp.float32) /
              jnp.full((16,), jnp.maximum(n_eq, 1), jnp.int32)
              .astype(jnp.float32))
        den_v = (jnp.full((16,), jnp.sum(dgt), jnp.float32)
                 + fv * jnp.full((16,), jnp.sum(deq), jnp.float32) + 1e-20)
        num_gt = jnp.where(lanes == 0, jnp.sum(ddt),
                           jnp.where(lanes == 1, jnp.sum(dht),
                                     jnp.where(lanes == 2, jnp.sum(dwt),
                                               0.0)))
        num_eq = jnp.where(lanes == 0, jnp.sum(det),
                           jnp.where(lanes == 1, jnp.sum(het),
                                     jnp.where(lanes == 2, jnp.sum(wet),
                                               0.0)))
        outv[...] = (num_gt + fv * num_eq) / den_v
        pltpu.sync_copy(outv, o_hbm.at[row])

    wid = lax.axis_index("s") * 2 + lax.axis_index("c")

    def rows(r, _):
        do_row(wid * 2 + r)
        return 0

    lax.fori_loop(0, 2, rows, jnp.int32(0))


def kernel(heatmap):
    B, C, D, H, W = heatmap.shape
    x = heatmap.reshape(B * C, _N)
    mesh = plsc.VectorSubcoreMesh(core_axis_name="c", subcore_axis_name="s")
    f = functools.partial(
        pl.kernel,
        mesh=mesh,
        out_type=jax.ShapeDtypeStruct((_ROWS, 16), jnp.float32),
        scratch_types=[
            pltpu.VMEM((_W,), jnp.float32),
            pltpu.VMEM((_CAP,), jnp.int32),
            pltpu.VMEM((_CAP,), jnp.int32),
            pltpu.VMEM((16,), jnp.float32),
            pltpu.SMEM((1,), jnp.int32),
            pltpu.SMEM((1,), jnp.int32),
            pltpu.SMEM((1,), jnp.float32),
            pltpu.SMEM((1,), jnp.float32),
        ],
        compiler_params=pltpu.CompilerParams(needs_layout_passes=False),
    )(_sc_kernel)
    out = f(x)
    return out[:, :3].reshape(B, C, 3)


# speedup vs baseline: 43.3339x; 1.1030x over previous
import functools

import jax
import jax.numpy as jnp
from jax import lax
from jax.experimental import pallas as pl
from jax.experimental.pallas import tpu as pltpu
from jax.experimental.pallas import tpu_sc as plsc

_TEMP = 0.5
_K = 256
_N = 96 * 96 * 96
_ROWS = 64
_W = 32768
_NWIN = _N // _W
_CHUNK = 512
_NCHUNK = _W // _CHUNK
_CAP = 2048
_IMIN = -2147483648


def _key_of(v):
    u = lax.bitcast_convert_type(v, jnp.int32)
    return jnp.where(u >= 0, u, u ^ jnp.int32(0x7FFFFFFF))


def _val_of_key(k):
    u = jnp.where(k >= 0, k, k ^ jnp.int32(0x7FFFFFFF))
    return lax.bitcast_convert_type(u, jnp.float32)


def _sc_kernel(x_hbm, o_hbm, win0, win1, ck, ci, outv,
               sptr, skey, sthr, smax, sem0, sem1):
    lanes = lax.iota(jnp.int32, 16)

    def count_ge(cand, ptr):
        nv = (ptr + 15) // 16

        def cbody(i, acc):
            kv = ck[pl.ds(i * 16, 16)]
            ok = jnp.logical_and(kv >= cand, i * 16 + lanes < ptr)
            return acc + jnp.where(ok, 1, 0).astype(jnp.int32)

        acc = lax.fori_loop(0, nv, cbody, jnp.zeros((16,), jnp.int32))
        return jnp.sum(acc)

    def kth_key(ptr, stop_cnt):
        def body(i, c):
            p, cnt = c

            def live(_):
                cand = p + (jnp.int32(1) << (31 - i))
                cn = count_ge(cand, ptr)
                take = cn >= _K
                return (jnp.where(take, cand, p), jnp.where(take, cn, cnt))

            done = jnp.logical_and(cnt >= _K, cnt <= stop_cnt)
            return lax.cond(done, lambda _: (p, cnt), live, 0)

        p, cnt = lax.fori_loop(0, 32, body,
                               (jnp.int32(_IMIN), jnp.int32(0x7FFFFFFF)))
        return p, cnt

    def reselect():
        ptr = sptr[0]
        tnew, _ = kth_key(ptr, 2 * _K)
        skey[0] = tnew
        sthr[0] = jnp.max(_val_of_key(jnp.full((16,), tnew, jnp.int32)))

        nv = (ptr + 15) // 16

        def comp(i, wp):
            kv = ck[pl.ds(i * 16, 16)]
            iv = ci[pl.ds(i * 16, 16)]
            msk = jnp.logical_and(kv >= tnew, i * 16 + lanes < ptr)
            plsc.store_compressed(ck.at[pl.ds(wp, 16)], kv, mask=msk)
            plsc.store_compressed(ci.at[pl.ds(wp, 16)], iv, mask=msk)
            return wp + jnp.sum(jnp.where(msk, 1, 0).astype(jnp.int32))

        sptr[0] = lax.fori_loop(0, nv, comp, jnp.int32(0))

    def process(win, w, row):
        def chunk(c, _):
            @pl.when(sptr[0] > _CAP - (_CHUNK + 8))
            def _():
                reselect()

            base = c * _CHUNK
            mv = win[pl.ds(base, 16)]
            for j in range(1, 32):
                mv = jnp.maximum(mv, win[pl.ds(base + j * 16, 16)])
            cmax = jnp.max(mv)
            smax[0] = jnp.maximum(smax[0], cmax)

            @pl.when(cmax >= sthr[0])
            def _():
                tkey = skey[0]
                ptr = sptr[0]
                gbase = w * _W + base
                for j in range(32):
                    v = win[pl.ds(base + j * 16, 16)]
                    kv = _key_of(v)
                    msk = kv >= tkey
                    plsc.store_compressed(ck.at[pl.ds(ptr, 16)], kv, mask=msk)
                    plsc.store_compressed(
                        ci.at[pl.ds(ptr, 16)], gbase + j * 16 + lanes,
                        mask=msk)
                    ptr = ptr + jnp.sum(
                        jnp.where(msk, 1, 0).astype(jnp.int32))
                sptr[0] = ptr
            return 0

        lax.fori_loop(0, _NCHUNK, chunk, jnp.int32(0))

    def do_row(row):
        sptr[0] = jnp.int32(0)
        skey[0] = jnp.int32(_IMIN)
        sthr[0] = jnp.float32(-jnp.inf)
        smax[0] = jnp.float32(-jnp.inf)

        pltpu.async_copy(x_hbm.at[row, pl.ds(0, _W)], win0, sem0)

        def window(w, _):
            def go(cur, cursem, nxt, nxtsem):
                pltpu.make_async_copy(
                    x_hbm.at[row, pl.ds(w * _W, _W)], cur, cursem).wait()

                @pl.when(w + 1 < _NWIN)
                def _():
                    pltpu.async_copy(
                        x_hbm.at[row, pl.ds((w + 1) * _W, _W)], nxt, nxtsem)

                process(cur, w, row)

            @pl.when(w % 2 == 0)
            def _():
                go(win0, sem0, win1, sem1)

            @pl.when(w % 2 == 1)
            def _():
                go(win1, sem1, win0, sem0)

            return 0

        lax.fori_loop(0, _NWIN, window, jnp.int32(0))

        ptr = sptr[0]
        tstar, _ = kth_key(ptr, _K)
        m = smax[0]
        nv = (ptr + 15) // 16

        def wbody(i, accs):
            dgt, ddt, dht, dwt, deq, det, het, wet, ngt, neq = accs
            kv = ck[pl.ds(i * 16, 16)]
            iv = ci[pl.ds(i * 16, 16)]
            occ = i * 16 + lanes < ptr
            gt = jnp.logical_and(kv > tstar, occ)
            eq = jnp.logical_and(kv == tstar, occ)
            ge = jnp.logical_or(gt, eq)
            vv = _val_of_key(kv)
            e = jnp.where(ge, jnp.exp((vv - m) * (1.0 / _TEMP)),
                          jnp.float32(0.0))
            d = (iv // 9216).astype(jnp.float32)
            rem = iv - (iv // 9216) * 9216
            h = (rem // 96).astype(jnp.float32)
            wc = (rem - (rem // 96) * 96).astype(jnp.float32)
            egt = jnp.where(gt, e, 0.0)
            eeq = jnp.where(eq, e, 0.0)
            return (dgt + egt, ddt + egt * d, dht + egt * h, dwt + egt * wc,
                    deq + eeq, det + eeq * d, het + eeq * h, wet + eeq * wc,
                    ngt + jnp.where(gt, 1, 0).astype(jnp.int32),
                    neq + jnp.where(eq, 1, 0).astype(jnp.int32))

        z = jnp.zeros((16,), jnp.float32)
        zi = jnp.zeros((16,), jnp.int32)
        accs = lax.fori_loop(0, nv, wbody,
                             (z, z, z, z, z, z, z, z, zi, zi))
        dgt, ddt, dht, dwt, deq, det, het, wet, ngt, neq = accs
        n_gt = jnp.sum(ngt)
        n_eq = jnp.sum(neq)
        fv = (jnp.full((16,), jnp.int32(_K) - n_gt, jnp.int32)
              .astype(jnp.float32) /
              jnp.full((16,), jnp.maximum(n_eq, 1), jnp.int32)
              .astype(jnp.float32))
        den_v = (jnp.full((16,), jnp.sum(dgt), jnp.float32)
                 + fv * jnp.full((16,), jnp.sum(deq), jnp.float32) + 1e-20)
        num_gt = jnp.where(lanes == 0, jnp.sum(ddt),
                           jnp.where(lanes == 1, jnp.sum(dht),
                                     jnp.where(lanes == 2, jnp.sum(dwt),
                                               0.0)))
        num_eq = jnp.where(lanes == 0, jnp.sum(det),
                           jnp.where(lanes == 1, jnp.sum(het),
                                     jnp.where(lanes == 2, jnp.sum(wet),
                                               0.0)))
        outv[...] = (num_gt + fv * num_eq) / den_v
        pltpu.sync_copy(outv, o_hbm.at[row])

    wid = lax.axis_index("s") * 2 + lax.axis_index("c")

    def rows(r, _):
        do_row(wid * 2 + r)
        return 0

    lax.fori_loop(0, 2, rows, jnp.int32(0))


def kernel(heatmap):
    B, C, D, H, W = heatmap.shape
    x = heatmap.reshape(B * C, _N)
    mesh = plsc.VectorSubcoreMesh(core_axis_name="c", subcore_axis_name="s")
    f = functools.partial(
        pl.kernel,
        mesh=mesh,
        out_type=jax.ShapeDtypeStruct((_ROWS, 16), jnp.float32),
        scratch_types=[
            pltpu.VMEM((_W,), jnp.float32),
            pltpu.VMEM((_W,), jnp.float32),
            pltpu.VMEM((_CAP,), jnp.int32),
            pltpu.VMEM((_CAP,), jnp.int32),
            pltpu.VMEM((16,), jnp.float32),
            pltpu.SMEM((1,), jnp.int32),
            pltpu.SMEM((1,), jnp.int32),
            pltpu.SMEM((1,), jnp.float32),
            pltpu.SMEM((1,), jnp.float32),
            pltpu.SemaphoreType.DMA,
            pltpu.SemaphoreType.DMA,
        ],
        compiler_params=pltpu.CompilerParams(needs_layout_passes=False),
    )(_sc_kernel)
    out = f(x)
    return out[:, :3].reshape(B, C, 3)
